# baseline (speedup 1.0000x reference)
import functools

import jax
import jax.numpy as jnp
from jax import lax
from jax.experimental import pallas as pl
from jax.experimental.pallas import tpu as pltpu
from jax.experimental.pallas import tpu_sc as plsc

N = 10000
NFEAT = 128
NHID = 128
NCLASS = 40
E = 320000
BN_EPS = 1e-5

NC = 2
NS = 16
NW = NC * NS
EPW = E // NW
NPAD = 10240
RPS = NPAD // NS

_PARAMS = {NFEAT: (80, 4, 2, True), NCLASS: (80, 6, 2, False)}
BC = 5


RB = 1280


def _mid_body(p_ref, w0_ref, b0_ref, gam_ref, bet_ref, mu_ref, var_ref,
              w2_ref, o_ref):
    agg = jnp.dot(p_ref[0] + p_ref[1], w0_ref[...],
                  preferred_element_type=jnp.float32,
                  precision=lax.Precision.HIGHEST) + b0_ref[...]
    scale = gam_ref[...] * lax.rsqrt(var_ref[...] + BN_EPS)
    shift = bet_ref[...] - mu_ref[...] * scale
    h = jnp.maximum(agg * scale + shift, 0.0)
    o_ref[...] = jnp.dot(h, w2_ref[...], preferred_element_type=jnp.float32,
                         precision=lax.Precision.HIGHEST)


def _fin_body(q_ref, b2_ref, o_ref):
    o_ref[...] = q_ref[0] + q_ref[1] + b2_ref[...]


def _vec(shape):
    return pl.BlockSpec(shape, lambda i: (0,) * len(shape))


_mid = pl.pallas_call(
    _mid_body,
    grid=(NPAD // RB,),
    in_specs=[
        pl.BlockSpec((2, RB, NFEAT), lambda i: (0, i, 0)),
        _vec((NFEAT, NHID)), _vec((NHID,)), _vec((NHID,)), _vec((NHID,)),
        _vec((NHID,)), _vec((NHID,)), _vec((NHID, NCLASS)),
    ],
    out_specs=pl.BlockSpec((RB, NCLASS), lambda i: (i, 0)),
    out_shape=jax.ShapeDtypeStruct((NPAD, NCLASS), jnp.float32))

_fin = pl.pallas_call(
    _fin_body,
    grid=(NPAD // RB,),
    in_specs=[
        pl.BlockSpec((2, RB, NCLASS), lambda i: (0, i, 0)),
        _vec((NCLASS,)),
    ],
    out_specs=pl.BlockSpec((RB, NCLASS), lambda i: (i, 0)),
    out_shape=jax.ShapeDtypeStruct((NPAD, NCLASS), jnp.float32))


@functools.cache
def _make_agg(D, interpret=False):
    K, NB, SD, stream_idx = _PARAMS[D]
    GA = NB - SD
    nchunk = EPW // K
    mesh = plsc.VectorSubcoreMesh(core_axis_name="c", subcore_axis_name="s",
                                  num_cores=NC, num_subcores=NS)
    if stream_idx:
        idx_shape = (3, BC, K)
        nblk = nchunk // BC
        assert GA <= BC
    else:
        idx_shape = (nchunk, K)

    @functools.partial(
        pl.kernel,
        out_type=jax.ShapeDtypeStruct((NC, NPAD, D), jnp.float32),
        mesh=mesh,
        scratch_types=[
            pltpu.VMEM_SHARED((NPAD, D), jnp.float32),
            pltpu.VMEM(idx_shape, jnp.int32),
            pltpu.VMEM(idx_shape, jnp.int32),
            pltpu.VMEM((NB, K, D), jnp.float32),
            pltpu.SemaphoreType.DMA,
            pltpu.SemaphoreType.DMA,
            pltpu.SemaphoreType.DMA,
        ],
        compiler_params=pltpu.CompilerParams(use_tc_tiling_on_sc=False),
        interpret=interpret,
    )
    def agg(h_hbm, src_hbm, dst_hbm, zeros_hbm, out_hbm,
            acc, src_v, dst_v, rows, gsem, ssem, isem):
        c = lax.axis_index("c")
        s = lax.axis_index("s")
        wid = c * NS + s
        off = pl.multiple_of(s * RPS, 8)
        pltpu.sync_copy(zeros_hbm, acc.at[pl.ds(off, RPS)])

        def scat_wait():
            pltpu.make_async_copy(rows.at[0], acc.at[pl.ds(0, K)],
                                  ssem).wait()

        if not stream_idx:
            pltpu.sync_copy(src_hbm.at[wid], src_v)
            pltpu.sync_copy(dst_hbm.at[wid], dst_v)
            plsc.subcore_barrier()
            for p in range(GA):
                pltpu.async_copy(h_hbm.at[src_v.at[p]], rows.at[p], gsem)

            def body(j, carry):
                b = lax.rem(j, NB)
                pltpu.make_async_copy(h_hbm.at[src_v.at[j]], rows.at[b],
                                      gsem).wait()

                @pl.when(j >= SD)
                def _():
                    scat_wait()

                @pl.when(j < nchunk - GA)
                def _():
                    pltpu.async_copy(h_hbm.at[src_v.at[j + GA]],
                                     rows.at[lax.rem(j + GA, NB)], gsem)

                pltpu.async_copy(rows.at[b], acc.at[dst_v.at[j]], ssem,
                                 add=True)
                return carry

            lax.fori_loop(0, nchunk, body, 0)
        else:
            pltpu.sync_copy(src_hbm.at[wid, pl.ds(0, BC)], src_v.at[0])
            pltpu.sync_copy(dst_hbm.at[wid, pl.ds(0, BC)], dst_v.at[0])
            plsc.subcore_barrier()
            for p in range(GA):
                pltpu.async_copy(h_hbm.at[src_v.at[0, p]], rows.at[p],
                                 gsem)
            pltpu.async_copy(src_hbm.at[wid, pl.ds(BC, BC)], src_v.at[1],
                             isem)
            pltpu.async_copy(dst_hbm.at[wid, pl.ds(BC, BC)], dst_v.at[1],
                             isem)

            def blk_body(blk, carry):
                pb = lax.rem(blk, 3)
                pbn = lax.rem(blk + 1, 3)
                j0 = blk * BC
                for r in range(BC):
                    j = j0 + r
                    b = lax.rem(j, NB)
                    pltpu.make_async_copy(h_hbm.at[src_v.at[pb, r]],
                                          rows.at[b], gsem).wait()

                    @pl.when(j >= SD)
                    def _():
                        scat_wait()

                    if r == BC - GA:
                        @pl.when(blk < nblk - 1)
                        def _():
                            pltpu.make_async_copy(
                                src_hbm.at[wid, pl.ds(0, BC)],
                                src_v.at[pbn], isem).wait()
                            pltpu.make_async_copy(
                                dst_hbm.at[wid, pl.ds(0, BC)],
                                dst_v.at[pbn], isem).wait()

                        @pl.when(blk < nblk - 2)
                        def _():
                            nxt = (blk + 2) * BC
                            pltpu.async_copy(
                                src_hbm.at[wid, pl.ds(nxt, BC)],
                                src_v.at[lax.rem(blk + 2, 3)], isem)
                            pltpu.async_copy(
                                dst_hbm.at[wid, pl.ds(nxt, BC)],
                                dst_v.at[lax.rem(blk + 2, 3)], isem)

                    if r + GA < BC:
                        gsrc = src_v.at[pb, r + GA]
                    else:
                        gsrc = src_v.at[pbn, r + GA - BC]

                    @pl.when(j < nchunk - GA)
                    def _():
                        pltpu.async_copy(h_hbm.at[gsrc],
                                         rows.at[lax.rem(j + GA, NB)],
                                         gsem)

                    pltpu.async_copy(rows.at[b], acc.at[dst_v.at[pb, r]],
                                     ssem, add=True)
                return carry

            lax.fori_loop(0, nblk, blk_body, 0)

        for _ in range(SD):
            scat_wait()
        plsc.subcore_barrier()
        pltpu.sync_copy(acc.at[pl.ds(off, RPS)],
                        out_hbm.at[c, pl.ds(off, RPS)])

    return agg


def kernel(x, edge_index, W0, b0, bn_gamma, bn_beta, bn_mean, bn_var, W2,
           b2):
    _agg_hid = _make_agg(NFEAT)
    _agg_cls = _make_agg(NCLASS)
    kh = _PARAMS[NFEAT][0]
    kc = _PARAMS[NCLASS][0]
    src_h = edge_index[0].reshape(NW, EPW // kh, kh)
    dst_h = edge_index[1].reshape(NW, EPW // kh, kh)
    src_c = edge_index[0].reshape(NW, EPW // kc, kc)
    dst_c = edge_index[1].reshape(NW, EPW // kc, kc)
    zeros_hid = jnp.zeros((RPS, NFEAT), jnp.float32)
    zeros_cls = jnp.zeros((RPS, NCLASS), jnp.float32)

    p1 = _agg_hid(x, src_h, dst_h, zeros_hid)
    h2 = _mid(p1, W0, b0, bn_gamma, bn_beta, bn_mean, bn_var, W2)
    p2 = _agg_cls(h2, src_c, dst_c, zeros_cls)
    return _fin(p2, b2)[:N]


# speedup vs baseline: 1.0458x; 1.0458x over previous
import functools

import jax
import jax.numpy as jnp
from jax import lax
from jax.experimental import pallas as pl
from jax.experimental.pallas import tpu as pltpu
from jax.experimental.pallas import tpu_sc as plsc

N = 10000
NFEAT = 128
NHID = 128
NCLASS = 40
E = 320000
BN_EPS = 1e-5

NC = 2
NS = 16
NW = NC * NS
EPW = E // NW
NPAD = 10240
RPS = NPAD // NS

_PARAMS = {NFEAT: (80, 4, 2, True), NCLASS: (80, 6, 2, False)}
BC = 5


RB = 1280


def _mid_body(p_ref, w0_ref, b0_ref, gam_ref, bet_ref, mu_ref, var_ref,
              w2_ref, o_ref):
    agg = jnp.dot(p_ref[0] + p_ref[1], w0_ref[...],
                  preferred_element_type=jnp.float32,
                  precision=lax.Precision.HIGHEST) + b0_ref[...]
    scale = gam_ref[...] * lax.rsqrt(var_ref[...] + BN_EPS)
    shift = bet_ref[...] - mu_ref[...] * scale
    h = jnp.maximum(agg * scale + shift, 0.0)
    o_ref[...] = jnp.dot(h, w2_ref[...], preferred_element_type=jnp.float32,
                         precision=lax.Precision.HIGHEST)


def _fin_body(q_ref, b2_ref, o_ref):
    o_ref[...] = q_ref[0] + q_ref[1] + b2_ref[...]


def _vec(shape):
    return pl.BlockSpec(shape, lambda i: (0,) * len(shape))


_mid = pl.pallas_call(
    _mid_body,
    grid=(NPAD // RB,),
    in_specs=[
        pl.BlockSpec((2, RB, NFEAT), lambda i: (0, i, 0)),
        _vec((NFEAT, NHID)), _vec((NHID,)), _vec((NHID,)), _vec((NHID,)),
        _vec((NHID,)), _vec((NHID,)), _vec((NHID, NCLASS)),
    ],
    out_specs=pl.BlockSpec((RB, NCLASS), lambda i: (i, 0)),
    out_shape=jax.ShapeDtypeStruct((NPAD, NCLASS), jnp.float32))

_fin = pl.pallas_call(
    _fin_body,
    grid=(NPAD // RB,),
    in_specs=[
        pl.BlockSpec((2, RB, NCLASS), lambda i: (0, i, 0)),
        _vec((NCLASS,)),
    ],
    out_specs=pl.BlockSpec((RB, NCLASS), lambda i: (i, 0)),
    out_shape=jax.ShapeDtypeStruct((NPAD, NCLASS), jnp.float32))


@functools.cache
def _make_agg(D, interpret=False):
    K, NB, SD, stream_idx = _PARAMS[D]
    GA = NB - SD
    nchunk = EPW // K
    mesh = plsc.VectorSubcoreMesh(core_axis_name="c", subcore_axis_name="s",
                                  num_cores=NC, num_subcores=NS)
    if stream_idx:
        idx_shape = (3, BC * K)
        nblk = nchunk // BC
        assert GA <= BC
    else:
        idx_shape = (EPW,)

    @functools.partial(
        pl.kernel,
        out_type=jax.ShapeDtypeStruct((NC, NPAD, D), jnp.float32),
        mesh=mesh,
        scratch_types=[
            pltpu.VMEM_SHARED((NPAD, D), jnp.float32),
            pltpu.VMEM(idx_shape, jnp.int32),
            pltpu.VMEM(idx_shape, jnp.int32),
            pltpu.VMEM((NB, K, D), jnp.float32),
            pltpu.SemaphoreType.DMA,
            pltpu.SemaphoreType.DMA,
            pltpu.SemaphoreType.DMA,
        ],
        compiler_params=pltpu.CompilerParams(use_tc_tiling_on_sc=False),
        interpret=interpret,
    )
    def agg(h_hbm, edge_hbm, zeros_hbm, out_hbm,
            acc, src_v, dst_v, rows, gsem, ssem, isem):
        c = lax.axis_index("c")
        s = lax.axis_index("s")
        wid = c * NS + s
        ebase = pl.multiple_of(wid * EPW, 8)
        off = pl.multiple_of(s * RPS, 8)
        pltpu.sync_copy(zeros_hbm, acc.at[pl.ds(off, RPS)])

        def scat_wait():
            pltpu.make_async_copy(rows.at[0], acc.at[pl.ds(0, K)],
                                  ssem).wait()

        if not stream_idx:
            pltpu.sync_copy(edge_hbm.at[0, pl.ds(ebase, EPW)], src_v)
            pltpu.sync_copy(edge_hbm.at[1, pl.ds(ebase, EPW)], dst_v)
            plsc.subcore_barrier()
            for p in range(GA):
                pltpu.async_copy(h_hbm.at[src_v.at[pl.ds(p * K, K)]],
                                 rows.at[p], gsem)

            def body(j, carry):
                b = lax.rem(j, NB)
                jK = pl.multiple_of(j * K, 8)
                pltpu.make_async_copy(h_hbm.at[src_v.at[pl.ds(jK, K)]],
                                      rows.at[b], gsem).wait()

                @pl.when(j >= SD)
                def _():
                    scat_wait()

                @pl.when(j < nchunk - GA)
                def _():
                    gK = pl.multiple_of((j + GA) * K, 8)
                    pltpu.async_copy(h_hbm.at[src_v.at[pl.ds(gK, K)]],
                                     rows.at[lax.rem(j + GA, NB)], gsem)

                pltpu.async_copy(rows.at[b],
                                 acc.at[dst_v.at[pl.ds(jK, K)]], ssem,
                                 add=True)
                return carry

            lax.fori_loop(0, nchunk, body, 0)
        else:
            BCK = BC * K
            pltpu.sync_copy(edge_hbm.at[0, pl.ds(ebase, BCK)], src_v.at[0])
            pltpu.sync_copy(edge_hbm.at[1, pl.ds(ebase, BCK)], dst_v.at[0])
            plsc.subcore_barrier()
            for p in range(GA):
                pltpu.async_copy(h_hbm.at[src_v.at[0, pl.ds(p * K, K)]],
                                 rows.at[p], gsem)
            pltpu.async_copy(edge_hbm.at[0, pl.ds(ebase + BCK, BCK)],
                             src_v.at[1], isem)
            pltpu.async_copy(edge_hbm.at[1, pl.ds(ebase + BCK, BCK)],
                             dst_v.at[1], isem)

            def blk_body(blk, carry):
                pb = lax.rem(blk, 3)
                pbn = lax.rem(blk + 1, 3)
                j0 = blk * BC
                for r in range(BC):
                    j = j0 + r
                    b = lax.rem(j, NB)
                    pltpu.make_async_copy(
                        h_hbm.at[src_v.at[pb, pl.ds(r * K, K)]],
                        rows.at[b], gsem).wait()

                    @pl.when(j >= SD)
                    def _():
                        scat_wait()

                    if r == BC - GA:
                        @pl.when(blk < nblk - 1)
                        def _():
                            pltpu.make_async_copy(
                                edge_hbm.at[0, pl.ds(ebase, BCK)],
                                src_v.at[pbn], isem).wait()
                            pltpu.make_async_copy(
                                edge_hbm.at[1, pl.ds(ebase, BCK)],
                                dst_v.at[pbn], isem).wait()

                        @pl.when(blk < nblk - 2)
                        def _():
                            nxt = ebase + (blk + 2) * BCK
                            pltpu.async_copy(
                                edge_hbm.at[0, pl.ds(nxt, BCK)],
                                src_v.at[lax.rem(blk + 2, 3)], isem)
                            pltpu.async_copy(
                                edge_hbm.at[1, pl.ds(nxt, BCK)],
                                dst_v.at[lax.rem(blk + 2, 3)], isem)

                    if r + GA < BC:
                        gsrc = src_v.at[pb, pl.ds((r + GA) * K, K)]
                    else:
                        gsrc = src_v.at[pbn, pl.ds((r + GA - BC) * K, K)]

                    @pl.when(j < nchunk - GA)
                    def _():
                        pltpu.async_copy(h_hbm.at[gsrc],
                                         rows.at[lax.rem(j + GA, NB)],
                                         gsem)

                    pltpu.async_copy(
                        rows.at[b],
                        acc.at[dst_v.at[pb, pl.ds(r * K, K)]], ssem,
                        add=True)
                return carry

            lax.fori_loop(0, nblk, blk_body, 0)

        for _ in range(SD):
            scat_wait()
        plsc.subcore_barrier()
        pltpu.sync_copy(acc.at[pl.ds(off, RPS)],
                        out_hbm.at[c, pl.ds(off, RPS)])

    return agg


def kernel(x, edge_index, W0, b0, bn_gamma, bn_beta, bn_mean, bn_var, W2,
           b2):
    _agg_hid = _make_agg(NFEAT)
    _agg_cls = _make_agg(NCLASS)
    zeros_hid = jnp.zeros((RPS, NFEAT), jnp.float32)
    zeros_cls = jnp.zeros((RPS, NCLASS), jnp.float32)

    p1 = _agg_hid(x, edge_index, zeros_hid)
    h2 = _mid(p1, W0, b0, bn_gamma, bn_beta, bn_mean, bn_var, W2)
    p2 = _agg_cls(h2, edge_index, zeros_cls)
    return _fin(p2, b2)[:N]


# speedup vs baseline: 1.0933x; 1.0454x over previous
import functools

import jax
import jax.numpy as jnp
from jax import lax
from jax.experimental import pallas as pl
from jax.experimental.pallas import tpu as pltpu
from jax.experimental.pallas import tpu_sc as plsc

N = 10000
NFEAT = 128
NHID = 128
NCLASS = 40
E = 320000
BN_EPS = 1e-5

NC = 2
NS = 16
NW = NC * NS
EPW = E // NW
NPAD = 10240
RPS = NPAD // NS

_PARAMS = {NFEAT: (80, 4, 2, True), NCLASS: (80, 6, 2, False)}
BC = 5


RB = 1280


def _mid_body(p_ref, w0_ref, b0_ref, gam_ref, bet_ref, mu_ref, var_ref,
              w2_ref, o_ref):
    agg = jnp.dot(p_ref[0] + p_ref[1], w0_ref[...],
                  preferred_element_type=jnp.float32) + b0_ref[...]
    scale = gam_ref[...] * lax.rsqrt(var_ref[...] + BN_EPS)
    shift = bet_ref[...] - mu_ref[...] * scale
    h = jnp.maximum(agg * scale + shift, 0.0)
    o_ref[...] = jnp.dot(h, w2_ref[...], preferred_element_type=jnp.float32)


def _fin_body(q_ref, b2_ref, o_ref):
    o_ref[...] = q_ref[0] + q_ref[1] + b2_ref[...]


def _vec(shape):
    return pl.BlockSpec(shape, lambda i: (0,) * len(shape))


_mid = pl.pallas_call(
    _mid_body,
    grid=(NPAD // RB,),
    in_specs=[
        pl.BlockSpec((2, RB, NFEAT), lambda i: (0, i, 0)),
        _vec((NFEAT, NHID)), _vec((NHID,)), _vec((NHID,)), _vec((NHID,)),
        _vec((NHID,)), _vec((NHID,)), _vec((NHID, NCLASS)),
    ],
    out_specs=pl.BlockSpec((RB, NCLASS), lambda i: (i, 0)),
    out_shape=jax.ShapeDtypeStruct((NPAD, NCLASS), jnp.float32))

_fin = pl.pallas_call(
    _fin_body,
    grid=(NPAD // RB,),
    in_specs=[
        pl.BlockSpec((2, RB, NCLASS), lambda i: (0, i, 0)),
        _vec((NCLASS,)),
    ],
    out_specs=pl.BlockSpec((RB, NCLASS), lambda i: (i, 0)),
    out_shape=jax.ShapeDtypeStruct((NPAD, NCLASS), jnp.float32))


@functools.cache
def _make_agg(D, interpret=False):
    K, NB, SD, stream_idx = _PARAMS[D]
    GA = NB - SD
    nchunk = EPW // K
    mesh = plsc.VectorSubcoreMesh(core_axis_name="c", subcore_axis_name="s",
                                  num_cores=NC, num_subcores=NS)
    if stream_idx:
        idx_shape = (3, BC * K)
        nblk = nchunk // BC
        assert GA <= BC
    else:
        idx_shape = (EPW,)

    @functools.partial(
        pl.kernel,
        out_type=jax.ShapeDtypeStruct((NC, NPAD, D), jnp.float32),
        mesh=mesh,
        scratch_types=[
            pltpu.VMEM_SHARED((NPAD, D), jnp.float32),
            pltpu.VMEM(idx_shape, jnp.int32),
            pltpu.VMEM(idx_shape, jnp.int32),
            pltpu.VMEM((NB, K, D), jnp.float32),
            pltpu.SemaphoreType.DMA,
            pltpu.SemaphoreType.DMA,
            pltpu.SemaphoreType.DMA,
        ],
        compiler_params=pltpu.CompilerParams(use_tc_tiling_on_sc=False),
        interpret=interpret,
    )
    def agg(h_hbm, edge_hbm, zeros_hbm, out_hbm,
            acc, src_v, dst_v, rows, gsem, ssem, isem):
        c = lax.axis_index("c")
        s = lax.axis_index("s")
        wid = c * NS + s
        ebase = pl.multiple_of(wid * EPW, 8)
        off = pl.multiple_of(s * RPS, 8)
        pltpu.sync_copy(zeros_hbm, acc.at[pl.ds(off, RPS)])

        def scat_wait():
            pltpu.make_async_copy(rows.at[0], acc.at[pl.ds(0, K)],
                                  ssem).wait()

        if not stream_idx:
            pltpu.sync_copy(edge_hbm.at[0, pl.ds(ebase, EPW)], src_v)
            pltpu.sync_copy(edge_hbm.at[1, pl.ds(ebase, EPW)], dst_v)
            plsc.subcore_barrier()
            for p in range(GA):
                pltpu.async_copy(h_hbm.at[src_v.at[pl.ds(p * K, K)]],
                                 rows.at[p], gsem)

            def body(j, carry):
                b = lax.rem(j, NB)
                jK = pl.multiple_of(j * K, 8)
                pltpu.make_async_copy(h_hbm.at[src_v.at[pl.ds(jK, K)]],
                                      rows.at[b], gsem).wait()

                @pl.when(j >= SD)
                def _():
                    scat_wait()

                @pl.when(j < nchunk - GA)
                def _():
                    gK = pl.multiple_of((j + GA) * K, 8)
                    pltpu.async_copy(h_hbm.at[src_v.at[pl.ds(gK, K)]],
                                     rows.at[lax.rem(j + GA, NB)], gsem)

                pltpu.async_copy(rows.at[b],
                                 acc.at[dst_v.at[pl.ds(jK, K)]], ssem,
                                 add=True)
                return carry

            lax.fori_loop(0, nchunk, body, 0)
        else:
            BCK = BC * K
            pltpu.sync_copy(edge_hbm.at[0, pl.ds(ebase, BCK)], src_v.at[0])
            pltpu.sync_copy(edge_hbm.at[1, pl.ds(ebase, BCK)], dst_v.at[0])
            plsc.subcore_barrier()
            for p in range(GA):
                pltpu.async_copy(h_hbm.at[src_v.at[0, pl.ds(p * K, K)]],
                                 rows.at[p], gsem)
            pltpu.async_copy(edge_hbm.at[0, pl.ds(ebase + BCK, BCK)],
                             src_v.at[1], isem)
            pltpu.async_copy(edge_hbm.at[1, pl.ds(ebase + BCK, BCK)],
                             dst_v.at[1], isem)

            def blk_body(blk, carry):
                pb = lax.rem(blk, 3)
                pbn = lax.rem(blk + 1, 3)
                j0 = blk * BC
                for r in range(BC):
                    j = j0 + r
                    b = lax.rem(j, NB)
                    pltpu.make_async_copy(
                        h_hbm.at[src_v.at[pb, pl.ds(r * K, K)]],
                        rows.at[b], gsem).wait()

                    @pl.when(j >= SD)
                    def _():
                        scat_wait()

                    if r == BC - GA:
                        @pl.when(blk < nblk - 1)
                        def _():
                            pltpu.make_async_copy(
                                edge_hbm.at[0, pl.ds(ebase, BCK)],
                                src_v.at[pbn], isem).wait()
                            pltpu.make_async_copy(
                                edge_hbm.at[1, pl.ds(ebase, BCK)],
                                dst_v.at[pbn], isem).wait()

                        @pl.when(blk < nblk - 2)
                        def _():
                            nxt = ebase + (blk + 2) * BCK
                            pltpu.async_copy(
                                edge_hbm.at[0, pl.ds(nxt, BCK)],
                                src_v.at[lax.rem(blk + 2, 3)], isem)
                            pltpu.async_copy(
                                edge_hbm.at[1, pl.ds(nxt, BCK)],
                                dst_v.at[lax.rem(blk + 2, 3)], isem)

                    if r + GA < BC:
                        gsrc = src_v.at[pb, pl.ds((r + GA) * K, K)]
                    else:
                        gsrc = src_v.at[pbn, pl.ds((r + GA - BC) * K, K)]

                    @pl.when(j < nchunk - GA)
                    def _():
                        pltpu.async_copy(h_hbm.at[gsrc],
                                         rows.at[lax.rem(j + GA, NB)],
                                         gsem)

                    pltpu.async_copy(
                        rows.at[b],
                        acc.at[dst_v.at[pb, pl.ds(r * K, K)]], ssem,
                        add=True)
                return carry

            lax.fori_loop(0, nblk, blk_body, 0)

        for _ in range(SD):
            scat_wait()
        plsc.subcore_barrier()
        pltpu.sync_copy(acc.at[pl.ds(off, RPS)],
                        out_hbm.at[c, pl.ds(off, RPS)])

    return agg


def kernel(x, edge_index, W0, b0, bn_gamma, bn_beta, bn_mean, bn_var, W2,
           b2):
    _agg_hid = _make_agg(NFEAT)
    _agg_cls = _make_agg(NCLASS)
    zeros_hid = jnp.zeros((RPS, NFEAT), jnp.float32)
    zeros_cls = jnp.zeros((RPS, NCLASS), jnp.float32)

    p1 = _agg_hid(x, edge_index, zeros_hid)
    h2 = _mid(p1, W0, b0, bn_gamma, bn_beta, bn_mean, bn_var, W2)
    p2 = _agg_cls(h2, edge_index, zeros_cls)
    return _fin(p2, b2)[:N]


# speedup vs baseline: 1.1214x; 1.0257x over previous
import functools

import jax
import jax.numpy as jnp
from jax import lax
from jax.experimental import pallas as pl
from jax.experimental.pallas import tpu as pltpu
from jax.experimental.pallas import tpu_sc as plsc

N = 10000
NFEAT = 128
NHID = 128
NCLASS = 40
E = 320000
BN_EPS = 1e-5

NC = 2
NS = 16
NW = NC * NS
EPW = E // NW
NPAD = 10240
RPS = NPAD // NS

_PARAMS = {NFEAT: (80, 4, 2, True), NCLASS: (80, 8, 3, False)}
BC = 5


RB = 1280


def _mid_body(p_ref, w0_ref, b0_ref, gam_ref, bet_ref, mu_ref, var_ref,
              w2_ref, o_ref):
    agg = jnp.dot(p_ref[0] + p_ref[1], w0_ref[...],
                  preferred_element_type=jnp.float32) + b0_ref[...]
    scale = gam_ref[...] * lax.rsqrt(var_ref[...] + BN_EPS)
    shift = bet_ref[...] - mu_ref[...] * scale
    h = jnp.maximum(agg * scale + shift, 0.0)
    o_ref[...] = jnp.dot(h, w2_ref[...], preferred_element_type=jnp.float32)


def _fin_body(q_ref, b2_ref, o_ref):
    o_ref[...] = q_ref[0] + q_ref[1] + b2_ref[...]


def _vec(shape):
    return pl.BlockSpec(shape, lambda i: (0,) * len(shape))


_mid = pl.pallas_call(
    _mid_body,
    grid=(NPAD // RB,),
    in_specs=[
        pl.BlockSpec((2, RB, NFEAT), lambda i: (0, i, 0)),
        _vec((NFEAT, NHID)), _vec((NHID,)), _vec((NHID,)), _vec((NHID,)),
        _vec((NHID,)), _vec((NHID,)), _vec((NHID, NCLASS)),
    ],
    out_specs=pl.BlockSpec((RB, NCLASS), lambda i: (i, 0)),
    out_shape=jax.ShapeDtypeStruct((NPAD, NCLASS), jnp.float32))

_fin = pl.pallas_call(
    _fin_body,
    grid=(NPAD // RB,),
    in_specs=[
        pl.BlockSpec((2, RB, NCLASS), lambda i: (0, i, 0)),
        _vec((NCLASS,)),
    ],
    out_specs=pl.BlockSpec((RB, NCLASS), lambda i: (i, 0)),
    out_shape=jax.ShapeDtypeStruct((NPAD, NCLASS), jnp.float32))


@functools.cache
def _make_agg(D, interpret=False):
    K, NB, SD, stream_idx = _PARAMS[D]
    GA = NB - SD
    nchunk = EPW // K
    mesh = plsc.VectorSubcoreMesh(core_axis_name="c", subcore_axis_name="s",
                                  num_cores=NC, num_subcores=NS)
    if stream_idx:
        idx_shape = (3, BC * K)
        nblk = nchunk // BC
        assert GA <= BC
    else:
        idx_shape = (EPW,)

    @functools.partial(
        pl.kernel,
        out_type=jax.ShapeDtypeStruct((NC, NPAD, D), jnp.float32),
        mesh=mesh,
        scratch_types=[
            pltpu.VMEM_SHARED((NPAD, D), jnp.float32),
            pltpu.VMEM(idx_shape, jnp.int32),
            pltpu.VMEM(idx_shape, jnp.int32),
            pltpu.VMEM((NB, K, D), jnp.float32),
            pltpu.SemaphoreType.DMA,
            pltpu.SemaphoreType.DMA,
            pltpu.SemaphoreType.DMA,
        ],
        compiler_params=pltpu.CompilerParams(use_tc_tiling_on_sc=False),
        interpret=interpret,
    )
    def agg(h_hbm, edge_hbm, zeros_hbm, out_hbm,
            acc, src_v, dst_v, rows, gsem, ssem, isem):
        c = lax.axis_index("c")
        s = lax.axis_index("s")
        wid = c * NS + s
        ebase = pl.multiple_of(wid * EPW, 8)
        off = pl.multiple_of(s * RPS, 8)
        pltpu.sync_copy(zeros_hbm, acc.at[pl.ds(off, RPS)])

        def scat_wait():
            pltpu.make_async_copy(rows.at[0], acc.at[pl.ds(0, K)],
                                  ssem).wait()

        if not stream_idx:
            pltpu.sync_copy(edge_hbm.at[0, pl.ds(ebase, EPW)], src_v)
            pltpu.sync_copy(edge_hbm.at[1, pl.ds(ebase, EPW)], dst_v)
            plsc.subcore_barrier()
            for p in range(GA):
                pltpu.async_copy(h_hbm.at[src_v.at[pl.ds(p * K, K)]],
                                 rows.at[p], gsem)

            def body(j, carry):
                b = lax.rem(j, NB)
                jK = pl.multiple_of(j * K, 8)
                pltpu.make_async_copy(h_hbm.at[src_v.at[pl.ds(jK, K)]],
                                      rows.at[b], gsem).wait()

                @pl.when(j >= SD)
                def _():
                    scat_wait()

                @pl.when(j < nchunk - GA)
                def _():
                    gK = pl.multiple_of((j + GA) * K, 8)
                    pltpu.async_copy(h_hbm.at[src_v.at[pl.ds(gK, K)]],
                                     rows.at[lax.rem(j + GA, NB)], gsem)

                pltpu.async_copy(rows.at[b],
                                 acc.at[dst_v.at[pl.ds(jK, K)]], ssem,
                                 add=True)
                return carry

            lax.fori_loop(0, nchunk, body, 0)
        else:
            BCK = BC * K
            pltpu.sync_copy(edge_hbm.at[0, pl.ds(ebase, BCK)], src_v.at[0])
            pltpu.sync_copy(edge_hbm.at[1, pl.ds(ebase, BCK)], dst_v.at[0])
            plsc.subcore_barrier()
            for p in range(GA):
                pltpu.async_copy(h_hbm.at[src_v.at[0, pl.ds(p * K, K)]],
                                 rows.at[p], gsem)
            pltpu.async_copy(edge_hbm.at[0, pl.ds(ebase + BCK, BCK)],
                             src_v.at[1], isem)
            pltpu.async_copy(edge_hbm.at[1, pl.ds(ebase + BCK, BCK)],
                             dst_v.at[1], isem)

            def blk_body(blk, carry):
                pb = lax.rem(blk, 3)
                pbn = lax.rem(blk + 1, 3)
                j0 = blk * BC
                for r in range(BC):
                    j = j0 + r
                    b = lax.rem(j, NB)
                    pltpu.make_async_copy(
                        h_hbm.at[src_v.at[pb, pl.ds(r * K, K)]],
                        rows.at[b], gsem).wait()

                    @pl.when(j >= SD)
                    def _():
                        scat_wait()

                    if r == BC - GA:
                        @pl.when(blk < nblk - 1)
                        def _():
                            pltpu.make_async_copy(
                                edge_hbm.at[0, pl.ds(ebase, BCK)],
                                src_v.at[pbn], isem).wait()
                            pltpu.make_async_copy(
                                edge_hbm.at[1, pl.ds(ebase, BCK)],
                                dst_v.at[pbn], isem).wait()

                        @pl.when(blk < nblk - 2)
                        def _():
                            nxt = ebase + (blk + 2) * BCK
                            pltpu.async_copy(
                                edge_hbm.at[0, pl.ds(nxt, BCK)],
                                src_v.at[lax.rem(blk + 2, 3)], isem)
                            pltpu.async_copy(
                                edge_hbm.at[1, pl.ds(nxt, BCK)],
                                dst_v.at[lax.rem(blk + 2, 3)], isem)

                    if r + GA < BC:
                        gsrc = src_v.at[pb, pl.ds((r + GA) * K, K)]
                    else:
                        gsrc = src_v.at[pbn, pl.ds((r + GA - BC) * K, K)]

                    @pl.when(j < nchunk - GA)
                    def _():
                        pltpu.async_copy(h_hbm.at[gsrc],
                                         rows.at[lax.rem(j + GA, NB)],
                                         gsem)

                    pltpu.async_copy(
                        rows.at[b],
                        acc.at[dst_v.at[pb, pl.ds(r * K, K)]], ssem,
                        add=True)
                return carry

            lax.fori_loop(0, nblk, blk_body, 0)

        for _ in range(SD):
            scat_wait()
        plsc.subcore_barrier()
        pltpu.sync_copy(acc.at[pl.ds(off, RPS)],
                        out_hbm.at[c, pl.ds(off, RPS)])

    return agg


def kernel(x, edge_index, W0, b0, bn_gamma, bn_beta, bn_mean, bn_var, W2,
           b2):
    _agg_hid = _make_agg(NFEAT)
    _agg_cls = _make_agg(NCLASS)
    zeros_hid = jnp.zeros((RPS, NFEAT), jnp.float32)
    zeros_cls = jnp.zeros((RPS, NCLASS), jnp.float32)

    p1 = _agg_hid(x, edge_index, zeros_hid)
    h2 = _mid(p1, W0, b0, bn_gamma, bn_beta, bn_mean, bn_var, W2)
    p2 = _agg_cls(h2, edge_index, zeros_cls)
    return _fin(p2, b2)[:N]
